# Initial kernel scaffold; baseline (speedup 1.0000x reference)
import functools

import jax
import jax.numpy as jnp
from jax import lax
from jax.experimental import pallas as pl
from jax.experimental.pallas import tpu as pltpu
from jax.experimental.pallas import tpu_sc as plsc

_B, _L, _D, _MAXS, _NSEG = 4, 4096, 768, 100, 50
_DS = 96
_NSTRIP = _D // _DS
_TCH = 256
_NCH = _L // _TCH
_LN = 16
_NROW = _NSEG + 1

_mesh = plsc.VectorSubcoreMesh(core_axis_name="c", subcore_axis_name="s")


@functools.partial(
    pl.kernel,
    out_type=jax.ShapeDtypeStruct((_B, _NSEG, _D), jnp.float32),
    mesh=_mesh,
    scratch_types=[
        pltpu.VMEM((_TCH, _DS), jnp.float32),
        pltpu.VMEM((_TCH,), jnp.int32),
        pltpu.VMEM((_NROW * _DS,), jnp.float32),
        pltpu.VMEM((_NSEG, _DS), jnp.float32),
    ],
)
def _sc_segmax(wf_hbm, ids_hbm, sf_hbm, x_v, ids_v, acc_v, o_v):
    cid = lax.axis_index("c")
    sid = lax.axis_index("s")
    wid = sid * 2 + cid
    b = wid // _NSTRIP
    d0 = (wid % _NSTRIP) * _DS

    neg_inf = jnp.full((_LN,), -jnp.inf, jnp.float32)

    def _init(i, carry):
        acc_v[pl.ds(i * _LN, _LN)] = neg_inf
        return carry

    lax.fori_loop(0, _NROW * _DS // _LN, _init, 0)

    lane = lax.iota(jnp.int32, _LN)

    def _chunk(c, carry):
        pltpu.sync_copy(wf_hbm.at[b, pl.ds(c * _TCH, _TCH), pl.ds(d0, _DS)],
                        x_v)
        pltpu.sync_copy(ids_hbm.at[b, pl.ds(c * _TCH, _TCH)], ids_v)

        def _group(g, carry2):
            ids16 = ids_v[pl.ds(g * _LN, _LN)]
            for j in range(_LN):
                t = g * _LN + j
                idsp = jnp.take(ids16, jnp.full((_LN,), j, jnp.int32),
                                mode="promise_in_bounds")
                for k in range(_DS // _LN):
                    addr = idsp * _DS + (k * _LN) + lane
                    xk = x_v[t, pl.ds(k * _LN, _LN)]
                    old = plsc.load_gather(acc_v, [addr])
                    plsc.store_scatter(acc_v, [addr], jnp.maximum(old, xk))
            return carry2

        lax.fori_loop(0, _TCH // _LN, _group, 0)
        return carry

    lax.fori_loop(0, _NCH, _chunk, 0)

    bb = jnp.max(ids_v[pl.ds(_TCH - _LN, _LN)])
    bbv = jnp.full((_LN,), bb, jnp.int32)
    zeros = jnp.zeros((_LN,), jnp.float32)
    for r in range(1, _NSEG + 1):
        valid = bbv >= r
        for k in range(_DS // _LN):
            v = acc_v[pl.ds(r * _DS + k * _LN, _LN)]
            o_v[r - 1, pl.ds(k * _LN, _LN)] = jnp.where(valid, v, zeros)

    pltpu.sync_copy(o_v, sf_hbm.at[b, :, pl.ds(d0, _DS)])


def _mm_body(x_ref, w_ref, b_ref, o_ref):
    y = lax.dot_general(x_ref[...], w_ref[...], (((1,), (1,)), ((), ())),
                        preferred_element_type=jnp.float32)
    y = y + b_ref[...]
    bias_rows = jnp.broadcast_to(b_ref[...], (_MAXS - _NSEG, _D))
    for i in range(_B):
        o_ref[pl.ds(i * _MAXS, _NSEG), :] = y[i * _NSEG:(i + 1) * _NSEG, :]
        o_ref[pl.ds(i * _MAXS + _NSEG, _MAXS - _NSEG), :] = bias_rows


_tc_linear = pl.pallas_call(
    _mm_body,
    out_shape=jax.ShapeDtypeStruct((_B * _MAXS, _D), jnp.float32),
)


def kernel(word_feature, sentence_mask, device, W, b):
    ids = sentence_mask.reshape(_B, _L).astype(jnp.int32)
    sf = _sc_segmax(word_feature, ids)
    out = _tc_linear(sf.reshape(_B * _NSEG, _D), W, b.reshape(1, _D))
    return out.reshape(_B, _MAXS, _D)


# speedup vs baseline: 2.2720x; 2.2720x over previous
import functools

import jax
import jax.numpy as jnp
from jax import lax
from jax.experimental import pallas as pl
from jax.experimental.pallas import tpu as pltpu
from jax.experimental.pallas import tpu_sc as plsc

_B, _L, _D, _MAXS, _NSEG = 4, 4096, 768, 100, 50
_NT = 8
_TPT = _L // _NT
_TCH = 64
_NCH = _TPT // _TCH
_LN = 16
_KV = _D // _LN
_NROW = _NSEG + 1

_mesh = plsc.VectorSubcoreMesh(core_axis_name="c", subcore_axis_name="s")


@functools.partial(
    pl.kernel,
    out_type=jax.ShapeDtypeStruct((_NT, _B, _NROW, _D), jnp.float32),
    mesh=_mesh,
    scratch_types=[
        pltpu.VMEM((_TCH, _D), jnp.float32),
        pltpu.VMEM((_TPT,), jnp.int32),
        pltpu.VMEM((_NROW, _D), jnp.float32),
    ],
    compiler_params=pltpu.CompilerParams(needs_layout_passes=False),
)
def _sc_segmax(wf_hbm, ids_hbm, part_hbm, x_v, ids_v, acc_v):
    cid = lax.axis_index("c")
    sid = lax.axis_index("s")
    wid = sid * 2 + cid
    b = wid // _NT
    e = wid % _NT
    t0 = e * _TPT

    pltpu.sync_copy(ids_hbm.at[b, pl.ds(t0, _TPT)], ids_v)

    neg_inf = jnp.full((_LN,), -jnp.inf, jnp.float32)
    lane = lax.iota(jnp.int32, _LN)

    def _init_row(i, carry):
        for k in range(_KV):
            acc_v[i, pl.ds(k * _LN, _LN)] = neg_inf
        return carry

    lax.fori_loop(0, _NROW, _init_row, 0)

    def _chunk(c, carry):
        pltpu.sync_copy(wf_hbm.at[b, pl.ds(t0 + c * _TCH, _TCH), :], x_v)

        def _tok(tl, carry2):
            tsp = jnp.full((_LN,), c * _TCH + tl, jnp.int32)
            idsp = plsc.load_gather(ids_v, [tsp])
            for k in range(_KV):
                col = k * _LN + lane
                xk = x_v[tl, pl.ds(k * _LN, _LN)]
                old = plsc.load_gather(acc_v, [idsp, col])
                plsc.store_scatter(acc_v, [idsp, col], jnp.maximum(old, xk))
            return carry2

        lax.fori_loop(0, _TCH, _tok, 0)
        return carry

    lax.fori_loop(0, _NCH, _chunk, 0)

    pltpu.sync_copy(acc_v, part_hbm.at[e, b])


def _tc_body(p_ref, w_ref, b_ref, v_ref, o_ref):
    w = w_ref[...]
    bias = b_ref[...]
    pad = jnp.broadcast_to(bias, (_MAXS - _NSEG, _D))
    for i in range(_B):
        m = p_ref[0, i, 1:, :]
        for e in range(1, _NT):
            m = jnp.maximum(m, p_ref[e, i, 1:, :])
        valid = v_ref[pl.ds(i * _NSEG, _NSEG), :] != 0
        m = jnp.where(valid, m, 0.0)
        y = lax.dot_general(m, w, (((1,), (1,)), ((), ())),
                            preferred_element_type=jnp.float32) + bias
        o_ref[pl.ds(i * _MAXS, _NSEG), :] = y
        o_ref[pl.ds(i * _MAXS + _NSEG, _MAXS - _NSEG), :] = pad


_tc_linear = pl.pallas_call(
    _tc_body,
    out_shape=jax.ShapeDtypeStruct((_B * _MAXS, _D), jnp.float32),
)


def kernel(word_feature, sentence_mask, device, W, b):
    ids = sentence_mask.reshape(_B, _L).astype(jnp.int32)
    part = _sc_segmax(word_feature, ids)
    bb = ids[:, -1]
    valid = (jnp.arange(1, _NSEG + 1)[None, :] <= bb[:, None])
    valid = valid.astype(jnp.int32).reshape(_B * _NSEG, 1)
    out = _tc_linear(part, W, b.reshape(1, _D), valid)
    return out.reshape(_B, _MAXS, _D)


# speedup vs baseline: 6.1352x; 2.7004x over previous
import functools

import jax
import jax.numpy as jnp
from jax import lax
from jax.experimental import pallas as pl
from jax.experimental.pallas import tpu as pltpu
from jax.experimental.pallas import tpu_sc as plsc

_B, _L, _D, _MAXS, _NSEG = 4, 4096, 768, 100, 50
_NT = 8
_TPT = _L // _NT
_TCH = 32
_NCH = _TPT // _TCH
_LN = 16
_GPC = _TCH // _LN
_KV = _D // _LN
_NROW = _NSEG + 1

_mesh = plsc.VectorSubcoreMesh(core_axis_name="c", subcore_axis_name="s")


@functools.partial(
    pl.kernel,
    out_type=jax.ShapeDtypeStruct((_NT, _B, _NROW, _D), jnp.float32),
    mesh=_mesh,
    scratch_types=[
        pltpu.VMEM((_TCH, _D), jnp.float32),
        pltpu.VMEM((_TCH, _D), jnp.float32),
        pltpu.VMEM((_TPT,), jnp.int32),
        pltpu.VMEM((_NROW, _D), jnp.float32),
        pltpu.SemaphoreType.DMA,
        pltpu.SemaphoreType.DMA,
    ],
    compiler_params=pltpu.CompilerParams(needs_layout_passes=False),
)
def _sc_segmax(wf_hbm, ids_hbm, part_hbm, x0_v, x1_v, ids_v, acc_v, sem0,
               sem1):
    cid = lax.axis_index("c")
    sid = lax.axis_index("s")
    wid = sid * 2 + cid
    b = wid // _NT
    e = wid % _NT
    t0 = e * _TPT

    pltpu.sync_copy(ids_hbm.at[b, pl.ds(t0, _TPT)], ids_v)

    neg_inf = jnp.full((_LN,), -jnp.inf, jnp.float32)
    bufs = (x0_v, x1_v)
    sems = (sem0, sem1)

    def _init_row(i, carry):
        for k in range(_KV):
            acc_v[i, pl.ds(k * _LN, _LN)] = neg_inf
        return carry

    lax.fori_loop(0, _NROW, _init_row, 0)

    def _start(c):
        return pltpu.async_copy(
            wf_hbm.at[b, pl.ds(t0 + c * _TCH, _TCH), :],
            bufs[c % 2], sems[c % 2])

    pending = _start(0)
    for c in range(_NCH):
        nxt = _start(c + 1) if c + 1 < _NCH else None
        pending.wait()
        x_v = bufs[c % 2]
        idg = [ids_v[pl.ds(c * _TCH + g * _LN, _LN)] for g in range(_GPC)]
        mn = jnp.min(idg[0])
        mx = jnp.max(idg[-1])

        def _seg(s, carry, idg=idg, x_v=x_v):
            sp = jnp.full((_LN,), s, jnp.int32)
            st = jnp.sum((idg[0] < sp).astype(jnp.int32))
            en = jnp.sum((idg[0] <= sp).astype(jnp.int32))
            for g in range(1, _GPC):
                st = st + jnp.sum((idg[g] < sp).astype(jnp.int32))
                en = en + jnp.sum((idg[g] <= sp).astype(jnp.int32))

            def _tok(t, accs, x_v=x_v):
                return tuple(
                    jnp.maximum(a, x_v[t, pl.ds(k * _LN, _LN)])
                    for k, a in enumerate(accs))

            accs = lax.fori_loop(st, en, _tok, (neg_inf,) * _KV)
            for k in range(_KV):
                col = pl.ds(k * _LN, _LN)
                acc_v[s, col] = jnp.maximum(acc_v[s, col], accs[k])
            return carry

        lax.fori_loop(mn, mx + 1, _seg, 0)
        pending = nxt

    pltpu.sync_copy(acc_v, part_hbm.at[e, b])


def _tc_body(p_ref, w_ref, b_ref, v_ref, o_ref):
    w = w_ref[...]
    bias = b_ref[...]
    pad = jnp.broadcast_to(bias, (_MAXS - _NSEG, _D))
    for i in range(_B):
        m = p_ref[0, i, 1:, :]
        for e in range(1, _NT):
            m = jnp.maximum(m, p_ref[e, i, 1:, :])
        valid = v_ref[pl.ds(i * _NSEG, _NSEG), :] != 0
        m = jnp.where(valid, m, 0.0)
        y = lax.dot_general(m, w, (((1,), (1,)), ((), ())),
                            preferred_element_type=jnp.float32) + bias
        o_ref[pl.ds(i * _MAXS, _NSEG), :] = y
        o_ref[pl.ds(i * _MAXS + _NSEG, _MAXS - _NSEG), :] = pad


_tc_linear = pl.pallas_call(
    _tc_body,
    out_shape=jax.ShapeDtypeStruct((_B * _MAXS, _D), jnp.float32),
)


def kernel(word_feature, sentence_mask, device, W, b):
    ids = sentence_mask.reshape(_B, _L).astype(jnp.int32)
    part = _sc_segmax(word_feature, ids)
    bb = ids[:, -1]
    valid = (jnp.arange(1, _NSEG + 1)[None, :] <= bb[:, None])
    valid = valid.astype(jnp.int32).reshape(_B * _NSEG, 1)
    out = _tc_linear(part, W, b.reshape(1, _D), valid)
    return out.reshape(_B, _MAXS, _D)
